# baseline (speedup 1.0000x reference)
import functools

import jax
import jax.numpy as jnp
from jax import lax
from jax.experimental import pallas as pl
from jax.experimental.pallas import tpu as pltpu
from jax.experimental.pallas import tpu_sc as plsc

_R = 524288
_NP = 32768
_B = 262144
_W = 16
_HP = _NP
_NTILES = 32
_CHUNK_R = _R // _NTILES
_CHUNK_B = _B // _NTILES
_KIDX = 128
_NSTREAM = _CHUNK_R // _KIDX
_SEG = 257 * 128
_BQ = 2048

_i32 = jnp.int32

_SC_PARAMS = pltpu.CompilerParams(needs_layout_passes=False)


_NBUF = 8


def _hist_body(preds_hbm, zeros_hbm, ones_hbm, out_hbm, idx_v, ones_v, hist_sh,
               sem):
    c = lax.axis_index("c")
    s = lax.axis_index("s")
    w = s * 2 + c

    @pl.when(s == 0)
    def _():
        pltpu.sync_copy(zeros_hbm, hist_sh)

    pltpu.sync_copy(preds_hbm.at[w], idx_v)
    pltpu.sync_copy(ones_hbm, ones_v)
    plsc.subcore_barrier()

    def body(j, carry):
        pltpu.async_copy(ones_v, hist_sh.at[idx_v.at[j]], sem, add=True)

        @pl.when(j >= _NBUF)
        def _():
            pltpu.make_async_copy(ones_v, hist_sh.at[idx_v.at[j]], sem).wait()

        return carry

    lax.fori_loop(0, _NSTREAM, body, 0)
    for _ in range(_NBUF):
        pltpu.make_async_copy(ones_v, hist_sh.at[idx_v.at[0]], sem).wait()
    plsc.subcore_barrier()

    @pl.when(s == 0)
    def _():
        pltpu.sync_copy(hist_sh, out_hbm.at[c])


def _hist_call(preds, zeros, ones):
    fn = pl.kernel(
        _hist_body,
        out_type=jax.ShapeDtypeStruct((2, _HP), _i32),
        mesh=plsc.VectorSubcoreMesh(core_axis_name="c", subcore_axis_name="s"),
        scratch_types=[
            pltpu.VMEM((_KIDX, _KIDX), _i32),
            pltpu.VMEM((_KIDX,), _i32),
            pltpu.VMEM_SHARED((_HP,), _i32),
            pltpu.SemaphoreType.DMA,
        ],
        compiler_params=_SC_PARAMS,
    )
    return fn(preds, zeros, ones)


def _scan_body(h_ref, seg_ref):
    h = h_ref[...]
    cnt = h[0] + h[1]
    x = cnt
    for k in (1, 2, 4, 8, 16, 32, 64):
        x = x + jnp.concatenate(
            [jnp.zeros((256, k), _i32), x[:, :-k]], axis=1)
    excl = jnp.concatenate(
        [jnp.zeros((256, 1), _i32), x[:, :-1]], axis=1)
    t = x[:, 127:128]
    for k in (1, 2, 4, 8, 16, 32, 64, 128):
        t = t + jnp.concatenate(
            [jnp.zeros((k, 1), _i32), t[:-k, :]], axis=0)
    rowpref = jnp.concatenate(
        [jnp.zeros((1, 1), _i32), t[:-1, :]], axis=0)
    seg = excl + rowpref
    total = t[255:256, :]
    last = jnp.broadcast_to(total, (1, 128))
    seg_ref[...] = jnp.concatenate([seg, last], axis=0)


def _scan_call(part):
    return pl.pallas_call(
        _scan_body,
        out_shape=jax.ShapeDtypeStruct((257, 128), _i32),
    )(part)


def _gather_body(seg_hbm, qp_hbm, starts_hbm, lens_hbm, seg_v, q_v, s_v, l_v,
                 sem):
    c = lax.axis_index("c")
    s = lax.axis_index("s")
    w = s * 2 + c
    seg_cp = pltpu.async_copy(seg_hbm, seg_v, sem)
    qp_cp = pltpu.async_copy(qp_hbm.at[w], q_v, sem)
    seg_cp.wait()
    qp_cp.wait()

    def body(i, carry):
        qp = jnp.clip(q_v[pl.ds(i * 16, 16)], 0, _NP)
        st = plsc.load_gather(seg_v, [qp])
        en = plsc.load_gather(seg_v, [qp + 1])
        s_v[pl.ds(i * 16, 16)] = st
        l_v[pl.ds(i * 16, 16)] = en - st
        return carry

    lax.fori_loop(0, _CHUNK_B // 16, body, 0, unroll=4)
    pltpu.sync_copy(s_v, starts_hbm.at[w])
    pltpu.sync_copy(l_v, lens_hbm.at[w])


def _gather_call(seg, qp2):
    fn = pl.kernel(
        _gather_body,
        out_type=(
            jax.ShapeDtypeStruct((_NTILES, _CHUNK_B), _i32),
            jax.ShapeDtypeStruct((_NTILES, _CHUNK_B), _i32),
        ),
        mesh=plsc.VectorSubcoreMesh(core_axis_name="c", subcore_axis_name="s"),
        scratch_types=[
            pltpu.VMEM((_SEG,), _i32),
            pltpu.VMEM((_CHUNK_B,), _i32),
            pltpu.VMEM((_CHUNK_B,), _i32),
            pltpu.VMEM((_CHUNK_B,), _i32),
            pltpu.SemaphoreType.DMA,
        ],
        compiler_params=_SC_PARAMS,
    )
    return fn(seg, qp2)


_BQ = 16384


def _expand_body(mp_ref, s_ref, l_ref, item_ref, valid_ref, qidx_ref):
    i = pl.program_id(0)
    s = s_ref[...]
    ln = jnp.minimum(l_ref[...], mp_ref[0, 0])
    row = lax.broadcasted_iota(_i32, (_W, _BQ), 0)
    item_ref[...] = s[None, :] + row
    valid_ref[...] = (row < ln[None, :]).astype(jnp.int8)
    qidx_ref[...] = i * _BQ + lax.broadcasted_iota(_i32, (_W, _BQ), 1)


def _expand_call(mp, starts, lens):
    return pl.pallas_call(
        _expand_body,
        grid=(_B // _BQ,),
        in_specs=[
            pl.BlockSpec(memory_space=pltpu.SMEM),
            pl.BlockSpec((_BQ,), lambda i: (i,)),
            pl.BlockSpec((_BQ,), lambda i: (i,)),
        ],
        out_specs=[
            pl.BlockSpec((_W, _BQ), lambda i: (0, i)),
            pl.BlockSpec((_W, _BQ), lambda i: (0, i)),
            pl.BlockSpec((_W, _BQ), lambda i: (0, i)),
        ],
        out_shape=[
            jax.ShapeDtypeStruct((_W, _B), _i32),
            jax.ShapeDtypeStruct((_W, _B), jnp.int8),
            jax.ShapeDtypeStruct((_W, _B), _i32),
        ],
    )(mp, starts, lens)


def kernel(rules_heads_idx, rules_bodies_idx, rule_lens, query_preds, max_pairs):
    preds = rules_heads_idx[:, 0].reshape(_NTILES, _KIDX, _KIDX)
    zeros = jnp.zeros((_HP,), _i32)
    ones = jnp.ones((_KIDX,), _i32)
    part = _hist_call(preds, zeros, ones)
    seg = _scan_call(part.reshape(2, 256, 128))
    qp2 = query_preds.reshape(_NTILES, _CHUNK_B)
    starts, lens = _gather_call(seg.reshape(_SEG), qp2)
    mp = jnp.asarray(max_pairs, _i32).reshape(1, 1)
    item_t, valid_t, qidx_t = _expand_call(
        mp, starts.reshape(_B), lens.reshape(_B))
    return (item_t.T, valid_t.astype(jnp.bool_).T, qidx_t.T)


# speedup vs baseline: 1.0448x; 1.0448x over previous
import functools

import jax
import jax.numpy as jnp
from jax import lax
from jax.experimental import pallas as pl
from jax.experimental.pallas import tpu as pltpu
from jax.experimental.pallas import tpu_sc as plsc

_R = 524288
_NP = 32768
_B = 262144
_W = 16
_HP = _NP
_NTILES = 32
_CHUNK_R = _R // _NTILES
_CHUNK_B = _B // _NTILES
_KIDX = 128
_NSTREAM = _CHUNK_R // _KIDX
_SEG = 257 * 128
_BQ = 2048

_i32 = jnp.int32

_SC_PARAMS = pltpu.CompilerParams(needs_layout_passes=False)


_NBUF = 8


def _hist_body(preds_hbm, zeros_hbm, ones_hbm, out_hbm, idx_v, ones_v, hist_sh,
               sem):
    c = lax.axis_index("c")
    s = lax.axis_index("s")
    w = s * 2 + c

    @pl.when(s == 0)
    def _():
        pltpu.sync_copy(zeros_hbm, hist_sh)

    pltpu.sync_copy(preds_hbm.at[w], idx_v)
    pltpu.sync_copy(ones_hbm, ones_v)
    plsc.subcore_barrier()

    def body(j, carry):
        pltpu.async_copy(ones_v, hist_sh.at[idx_v.at[j]], sem, add=True)

        @pl.when(j >= _NBUF)
        def _():
            pltpu.make_async_copy(ones_v, hist_sh.at[idx_v.at[j]], sem).wait()

        return carry

    lax.fori_loop(0, _NSTREAM, body, 0)
    for _ in range(_NBUF):
        pltpu.make_async_copy(ones_v, hist_sh.at[idx_v.at[0]], sem).wait()
    plsc.subcore_barrier()

    @pl.when(s == 0)
    def _():
        pltpu.sync_copy(hist_sh, out_hbm.at[c])


def _hist_call(preds, zeros, ones):
    fn = pl.kernel(
        _hist_body,
        out_type=jax.ShapeDtypeStruct((2, _HP), _i32),
        mesh=plsc.VectorSubcoreMesh(core_axis_name="c", subcore_axis_name="s"),
        scratch_types=[
            pltpu.VMEM((_KIDX, _KIDX), _i32),
            pltpu.VMEM((_KIDX,), _i32),
            pltpu.VMEM_SHARED((_HP,), _i32),
            pltpu.SemaphoreType.DMA,
        ],
        compiler_params=_SC_PARAMS,
    )
    return fn(preds, zeros, ones)


def _scan_body(h_ref, seg_ref):
    h = h_ref[...]
    cnt = h[0] + h[1]
    x = cnt
    for k in (1, 2, 4, 8, 16, 32, 64):
        x = x + jnp.concatenate(
            [jnp.zeros((256, k), _i32), x[:, :-k]], axis=1)
    excl = jnp.concatenate(
        [jnp.zeros((256, 1), _i32), x[:, :-1]], axis=1)
    t = x[:, 127:128]
    for k in (1, 2, 4, 8, 16, 32, 64, 128):
        t = t + jnp.concatenate(
            [jnp.zeros((k, 1), _i32), t[:-k, :]], axis=0)
    rowpref = jnp.concatenate(
        [jnp.zeros((1, 1), _i32), t[:-1, :]], axis=0)
    seg = excl + rowpref
    total = t[255:256, :]
    last = jnp.broadcast_to(total, (1, 128))
    seg_ref[...] = jnp.concatenate([seg, last], axis=0)


def _scan_call(part):
    return pl.pallas_call(
        _scan_body,
        out_shape=jax.ShapeDtypeStruct((257, 128), _i32),
    )(part)


def _gather_body(seg_hbm, qp_hbm, starts_hbm, lens_hbm, seg_v, q_v, s_v, l_v,
                 sem):
    c = lax.axis_index("c")
    s = lax.axis_index("s")
    w = s * 2 + c
    seg_cp = pltpu.async_copy(seg_hbm, seg_v, sem)
    qp_cp = pltpu.async_copy(qp_hbm.at[w], q_v, sem)
    seg_cp.wait()
    qp_cp.wait()

    def body(i, carry):
        qp = jnp.clip(q_v[pl.ds(i * 16, 16)], 0, _NP)
        st = plsc.load_gather(seg_v, [qp])
        en = plsc.load_gather(seg_v, [qp + 1])
        s_v[pl.ds(i * 16, 16)] = st
        l_v[pl.ds(i * 16, 16)] = en - st
        return carry

    lax.fori_loop(0, _CHUNK_B // 16, body, 0)
    pltpu.sync_copy(s_v, starts_hbm.at[w])
    pltpu.sync_copy(l_v, lens_hbm.at[w])


def _gather_call(seg, qp2):
    fn = pl.kernel(
        _gather_body,
        out_type=(
            jax.ShapeDtypeStruct((_NTILES, _CHUNK_B), _i32),
            jax.ShapeDtypeStruct((_NTILES, _CHUNK_B), _i32),
        ),
        mesh=plsc.VectorSubcoreMesh(core_axis_name="c", subcore_axis_name="s"),
        scratch_types=[
            pltpu.VMEM((_SEG,), _i32),
            pltpu.VMEM((_CHUNK_B,), _i32),
            pltpu.VMEM((_CHUNK_B,), _i32),
            pltpu.VMEM((_CHUNK_B,), _i32),
            pltpu.SemaphoreType.DMA,
        ],
        compiler_params=_SC_PARAMS,
    )
    return fn(seg, qp2)


_BQ = 16384


def _expand_body(mp_ref, s_ref, l_ref, item_ref, valid_ref, qidx_ref):
    i = pl.program_id(0)
    s = s_ref[...]
    ln = jnp.minimum(l_ref[...], mp_ref[0, 0])
    row = lax.broadcasted_iota(_i32, (_W, _BQ), 0)
    item_ref[...] = s[None, :] + row
    valid_ref[...] = (row < ln[None, :]).astype(jnp.int8)
    qidx_ref[...] = i * _BQ + lax.broadcasted_iota(_i32, (_W, _BQ), 1)


def _expand_call(mp, starts, lens):
    return pl.pallas_call(
        _expand_body,
        grid=(_B // _BQ,),
        in_specs=[
            pl.BlockSpec(memory_space=pltpu.SMEM),
            pl.BlockSpec((_BQ,), lambda i: (i,)),
            pl.BlockSpec((_BQ,), lambda i: (i,)),
        ],
        out_specs=[
            pl.BlockSpec((_W, _BQ), lambda i: (0, i)),
            pl.BlockSpec((_W, _BQ), lambda i: (0, i)),
            pl.BlockSpec((_W, _BQ), lambda i: (0, i)),
        ],
        out_shape=[
            jax.ShapeDtypeStruct((_W, _B), _i32),
            jax.ShapeDtypeStruct((_W, _B), jnp.int8),
            jax.ShapeDtypeStruct((_W, _B), _i32),
        ],
    )(mp, starts, lens)


def kernel(rules_heads_idx, rules_bodies_idx, rule_lens, query_preds, max_pairs):
    preds = rules_heads_idx[:, 0].reshape(_NTILES, _KIDX, _KIDX)
    zeros = jnp.zeros((_HP,), _i32)
    ones = jnp.ones((_KIDX,), _i32)
    part = _hist_call(preds, zeros, ones)
    seg = _scan_call(part.reshape(2, 256, 128))
    qp2 = query_preds.reshape(_NTILES, _CHUNK_B)
    starts, lens = _gather_call(seg.reshape(_SEG), qp2)
    mp = jnp.asarray(max_pairs, _i32).reshape(1, 1)
    item_t, valid_t, qidx_t = _expand_call(
        mp, starts.reshape(_B), lens.reshape(_B))
    return (item_t.T, valid_t.astype(jnp.bool_).T, qidx_t.T)


# speedup vs baseline: 1.0865x; 1.0399x over previous
import functools

import jax
import jax.numpy as jnp
from jax import lax
from jax.experimental import pallas as pl
from jax.experimental.pallas import tpu as pltpu
from jax.experimental.pallas import tpu_sc as plsc

_R = 524288
_NP = 32768
_B = 262144
_W = 16
_HP = _NP
_NTILES = 32
_CHUNK_R = _R // _NTILES
_CHUNK_B = _B // _NTILES
_KIDX = 128
_NSTREAM = _CHUNK_R // _KIDX
_SEG = 257 * 128
_BQ = 2048

_i32 = jnp.int32

_SC_PARAMS = pltpu.CompilerParams(needs_layout_passes=False)


_NBUF = 16


def _hist_body(preds_hbm, zeros_hbm, ones_hbm, out_hbm, idx_v, ones_v, hist_sh,
               sem):
    c = lax.axis_index("c")
    s = lax.axis_index("s")
    w = s * 2 + c

    @pl.when(s == 0)
    def _():
        pltpu.sync_copy(zeros_hbm, hist_sh)

    pltpu.sync_copy(preds_hbm.at[w], idx_v)
    pltpu.sync_copy(ones_hbm, ones_v)
    plsc.subcore_barrier()

    def body(j, carry):
        pltpu.async_copy(ones_v, hist_sh.at[idx_v.at[j]], sem, add=True)

        @pl.when(j >= _NBUF)
        def _():
            pltpu.make_async_copy(ones_v, hist_sh.at[idx_v.at[j]], sem).wait()

        return carry

    lax.fori_loop(0, _NSTREAM, body, 0)
    for _ in range(_NBUF):
        pltpu.make_async_copy(ones_v, hist_sh.at[idx_v.at[0]], sem).wait()
    plsc.subcore_barrier()

    @pl.when(s == 0)
    def _():
        pltpu.sync_copy(hist_sh, out_hbm.at[c])


def _hist_call(preds, zeros, ones):
    fn = pl.kernel(
        _hist_body,
        out_type=jax.ShapeDtypeStruct((2, _HP), _i32),
        mesh=plsc.VectorSubcoreMesh(core_axis_name="c", subcore_axis_name="s"),
        scratch_types=[
            pltpu.VMEM((_KIDX, _KIDX), _i32),
            pltpu.VMEM((_KIDX,), _i32),
            pltpu.VMEM_SHARED((_HP,), _i32),
            pltpu.SemaphoreType.DMA,
        ],
        compiler_params=_SC_PARAMS,
    )
    return fn(preds, zeros, ones)


def _scan_body(h_ref, seg_ref):
    h = h_ref[...]
    cnt = h[0] + h[1]
    x = cnt
    for k in (1, 2, 4, 8, 16, 32, 64):
        x = x + jnp.concatenate(
            [jnp.zeros((256, k), _i32), x[:, :-k]], axis=1)
    excl = jnp.concatenate(
        [jnp.zeros((256, 1), _i32), x[:, :-1]], axis=1)
    t = x[:, 127:128]
    for k in (1, 2, 4, 8, 16, 32, 64, 128):
        t = t + jnp.concatenate(
            [jnp.zeros((k, 1), _i32), t[:-k, :]], axis=0)
    rowpref = jnp.concatenate(
        [jnp.zeros((1, 1), _i32), t[:-1, :]], axis=0)
    seg = excl + rowpref
    total = t[255:256, :]
    last = jnp.broadcast_to(total, (1, 128))
    seg_ref[...] = jnp.concatenate([seg, last], axis=0)


def _scan_call(part):
    return pl.pallas_call(
        _scan_body,
        out_shape=jax.ShapeDtypeStruct((257, 128), _i32),
    )(part)


def _gather_body(seg_hbm, qp_hbm, starts_hbm, lens_hbm, seg_v, q_v, s_v, l_v,
                 sem):
    c = lax.axis_index("c")
    s = lax.axis_index("s")
    w = s * 2 + c
    seg_cp = pltpu.async_copy(seg_hbm, seg_v, sem)
    qp_cp = pltpu.async_copy(qp_hbm.at[w], q_v, sem)
    seg_cp.wait()
    qp_cp.wait()

    def body(i, carry):
        qp = jnp.clip(q_v[pl.ds(i * 16, 16)], 0, _NP)
        st = plsc.load_gather(seg_v, [qp])
        en = plsc.load_gather(seg_v, [qp + 1])
        s_v[pl.ds(i * 16, 16)] = st
        l_v[pl.ds(i * 16, 16)] = en - st
        return carry

    lax.fori_loop(0, _CHUNK_B // 16, body, 0)
    pltpu.sync_copy(s_v, starts_hbm.at[w])
    pltpu.sync_copy(l_v, lens_hbm.at[w])


def _gather_call(seg, qp2):
    fn = pl.kernel(
        _gather_body,
        out_type=(
            jax.ShapeDtypeStruct((_NTILES, _CHUNK_B), _i32),
            jax.ShapeDtypeStruct((_NTILES, _CHUNK_B), _i32),
        ),
        mesh=plsc.VectorSubcoreMesh(core_axis_name="c", subcore_axis_name="s"),
        scratch_types=[
            pltpu.VMEM((_SEG,), _i32),
            pltpu.VMEM((_CHUNK_B,), _i32),
            pltpu.VMEM((_CHUNK_B,), _i32),
            pltpu.VMEM((_CHUNK_B,), _i32),
            pltpu.SemaphoreType.DMA,
        ],
        compiler_params=_SC_PARAMS,
    )
    return fn(seg, qp2)


_BQ = 32768


def _expand_body(mp_ref, s_ref, l_ref, item_ref, valid_ref, qidx_ref):
    i = pl.program_id(0)
    s = s_ref[...]
    ln = jnp.minimum(l_ref[...], mp_ref[0, 0])
    row = lax.broadcasted_iota(_i32, (_W, _BQ), 0)
    item_ref[...] = s[None, :] + row
    valid_ref[...] = (row < ln[None, :]).astype(jnp.int8)
    qidx_ref[...] = i * _BQ + lax.broadcasted_iota(_i32, (_W, _BQ), 1)


def _expand_call(mp, starts, lens):
    return pl.pallas_call(
        _expand_body,
        grid=(_B // _BQ,),
        in_specs=[
            pl.BlockSpec(memory_space=pltpu.SMEM),
            pl.BlockSpec((_BQ,), lambda i: (i,)),
            pl.BlockSpec((_BQ,), lambda i: (i,)),
        ],
        out_specs=[
            pl.BlockSpec((_W, _BQ), lambda i: (0, i)),
            pl.BlockSpec((_W, _BQ), lambda i: (0, i)),
            pl.BlockSpec((_W, _BQ), lambda i: (0, i)),
        ],
        out_shape=[
            jax.ShapeDtypeStruct((_W, _B), _i32),
            jax.ShapeDtypeStruct((_W, _B), jnp.int8),
            jax.ShapeDtypeStruct((_W, _B), _i32),
        ],
    )(mp, starts, lens)


def kernel(rules_heads_idx, rules_bodies_idx, rule_lens, query_preds, max_pairs):
    preds = rules_heads_idx[:, 0].reshape(_NTILES, _KIDX, _KIDX)
    zeros = jnp.zeros((_HP,), _i32)
    ones = jnp.ones((_KIDX,), _i32)
    part = _hist_call(preds, zeros, ones)
    seg = _scan_call(part.reshape(2, 256, 128))
    qp2 = query_preds.reshape(_NTILES, _CHUNK_B)
    starts, lens = _gather_call(seg.reshape(_SEG), qp2)
    mp = jnp.asarray(max_pairs, _i32).reshape(1, 1)
    item_t, valid_t, qidx_t = _expand_call(
        mp, starts.reshape(_B), lens.reshape(_B))
    return (item_t.T, valid_t.astype(jnp.bool_).T, qidx_t.T)


# speedup vs baseline: 1.0879x; 1.0013x over previous
import jax
import jax.numpy as jnp
from jax import lax
from jax.experimental import pallas as pl
from jax.experimental.pallas import tpu as pltpu
from jax.experimental.pallas import tpu_sc as plsc

_R = 524288
_NP = 32768
_B = 262144
_W = 16
_HP = _NP
_NTILES = 32
_CHUNK_R = _R // _NTILES
_CHUNK_B = _B // _NTILES
_KIDX = 128
_NSTREAM = _CHUNK_R // _KIDX
_SEG = 257 * 128

_i32 = jnp.int32

_SC_PARAMS = pltpu.CompilerParams(needs_layout_passes=False)


_NBUF = 16


def _hist_body(preds_hbm, zeros_hbm, ones_hbm, out_hbm, idx_v, ones_v, hist_sh,
               sem):
    c = lax.axis_index("c")
    s = lax.axis_index("s")
    w = s * 2 + c

    @pl.when(s == 0)
    def _():
        pltpu.sync_copy(zeros_hbm, hist_sh)

    pltpu.sync_copy(preds_hbm.at[w], idx_v)
    pltpu.sync_copy(ones_hbm, ones_v)
    plsc.subcore_barrier()

    def body(j, carry):
        pltpu.async_copy(ones_v, hist_sh.at[idx_v.at[j]], sem, add=True)

        @pl.when(j >= _NBUF)
        def _():
            pltpu.make_async_copy(ones_v, hist_sh.at[idx_v.at[j]], sem).wait()

        return carry

    lax.fori_loop(0, _NSTREAM, body, 0)
    for _ in range(_NBUF):
        pltpu.make_async_copy(ones_v, hist_sh.at[idx_v.at[0]], sem).wait()
    plsc.subcore_barrier()

    @pl.when(s == 0)
    def _():
        pltpu.sync_copy(hist_sh, out_hbm.at[c])


def _hist_call(preds, zeros, ones):
    fn = pl.kernel(
        _hist_body,
        out_type=jax.ShapeDtypeStruct((2, _HP), _i32),
        mesh=plsc.VectorSubcoreMesh(core_axis_name="c", subcore_axis_name="s"),
        scratch_types=[
            pltpu.VMEM((_KIDX, _KIDX), _i32),
            pltpu.VMEM((_KIDX,), _i32),
            pltpu.VMEM_SHARED((_HP,), _i32),
            pltpu.SemaphoreType.DMA,
        ],
        compiler_params=_SC_PARAMS,
    )
    return fn(preds, zeros, ones)


def _scan_body(h_ref, seg_ref):
    h = h_ref[...]
    cnt = h[0] + h[1]
    x = cnt
    for k in (1, 2, 4, 8, 16, 32, 64):
        x = x + jnp.concatenate(
            [jnp.zeros((256, k), _i32), x[:, :-k]], axis=1)
    excl = jnp.concatenate(
        [jnp.zeros((256, 1), _i32), x[:, :-1]], axis=1)
    t = x[:, 127:128]
    for k in (1, 2, 4, 8, 16, 32, 64, 128):
        t = t + jnp.concatenate(
            [jnp.zeros((k, 1), _i32), t[:-k, :]], axis=0)
    rowpref = jnp.concatenate(
        [jnp.zeros((1, 1), _i32), t[:-1, :]], axis=0)
    seg = excl + rowpref
    total = t[255:256, :]
    last = jnp.broadcast_to(total, (1, 128))
    seg_ref[...] = jnp.concatenate([seg, last], axis=0)


def _scan_call(part):
    return pl.pallas_call(
        _scan_body,
        out_shape=jax.ShapeDtypeStruct((257, 128), _i32),
    )(part)


def _gather_body(seg_hbm, qp_hbm, starts_hbm, lens_hbm, seg_v, q_v, s_v, l_v,
                 sem):
    c = lax.axis_index("c")
    s = lax.axis_index("s")
    w = s * 2 + c
    seg_cp = pltpu.async_copy(seg_hbm, seg_v, sem)
    qp_cp = pltpu.async_copy(qp_hbm.at[w], q_v, sem)
    seg_cp.wait()
    qp_cp.wait()

    def body(i, carry):
        qp = jnp.clip(q_v[pl.ds(i * 16, 16)], 0, _NP)
        st = plsc.load_gather(seg_v, [qp])
        en = plsc.load_gather(seg_v, [qp + 1])
        s_v[pl.ds(i * 16, 16)] = st
        l_v[pl.ds(i * 16, 16)] = en - st
        return carry

    lax.fori_loop(0, _CHUNK_B // 16, body, 0)
    pltpu.sync_copy(s_v, starts_hbm.at[w])
    pltpu.sync_copy(l_v, lens_hbm.at[w])


def _gather_call(seg, qp2):
    fn = pl.kernel(
        _gather_body,
        out_type=(
            jax.ShapeDtypeStruct((_NTILES, _CHUNK_B), _i32),
            jax.ShapeDtypeStruct((_NTILES, _CHUNK_B), _i32),
        ),
        mesh=plsc.VectorSubcoreMesh(core_axis_name="c", subcore_axis_name="s"),
        scratch_types=[
            pltpu.VMEM((_SEG,), _i32),
            pltpu.VMEM((_CHUNK_B,), _i32),
            pltpu.VMEM((_CHUNK_B,), _i32),
            pltpu.VMEM((_CHUNK_B,), _i32),
            pltpu.SemaphoreType.DMA,
        ],
        compiler_params=_SC_PARAMS,
    )
    return fn(seg, qp2)


_BQ = 32768


def _expand_body(mp_ref, s_ref, l_ref, item_ref, valid_ref, qidx_ref):
    i = pl.program_id(0)
    s = s_ref[...]
    ln = jnp.minimum(l_ref[...], mp_ref[0, 0])
    row = lax.broadcasted_iota(_i32, (_W, _BQ), 0)
    item_ref[...] = s[None, :] + row
    valid_ref[...] = (row < ln[None, :]).astype(jnp.int8)
    qidx_ref[...] = i * _BQ + lax.broadcasted_iota(_i32, (_W, _BQ), 1)


def _expand_call(mp, starts, lens):
    return pl.pallas_call(
        _expand_body,
        grid=(_B // _BQ,),
        in_specs=[
            pl.BlockSpec(memory_space=pltpu.SMEM),
            pl.BlockSpec((_BQ,), lambda i: (i,)),
            pl.BlockSpec((_BQ,), lambda i: (i,)),
        ],
        out_specs=[
            pl.BlockSpec((_W, _BQ), lambda i: (0, i)),
            pl.BlockSpec((_W, _BQ), lambda i: (0, i)),
            pl.BlockSpec((_W, _BQ), lambda i: (0, i)),
        ],
        out_shape=[
            jax.ShapeDtypeStruct((_W, _B), _i32),
            jax.ShapeDtypeStruct((_W, _B), jnp.int8),
            jax.ShapeDtypeStruct((_W, _B), _i32),
        ],
    )(mp, starts, lens)


def kernel(rules_heads_idx, rules_bodies_idx, rule_lens, query_preds, max_pairs):
    preds = rules_heads_idx[:, 0].reshape(_NTILES, _KIDX, _KIDX)
    zeros = jnp.zeros((_HP,), _i32)
    ones = jnp.ones((_KIDX,), _i32)
    part = _hist_call(preds, zeros, ones)
    seg = _scan_call(part.reshape(2, 256, 128))
    qp2 = query_preds.reshape(_NTILES, _CHUNK_B)
    starts, lens = _gather_call(seg.reshape(_SEG), qp2)
    mp = jnp.asarray(max_pairs, _i32).reshape(1, 1)
    item_t, valid_t, qidx_t = _expand_call(
        mp, starts.reshape(_B), lens.reshape(_B))
    return (item_t.T, valid_t.astype(jnp.bool_).T, qidx_t.T)
